# Initial kernel scaffold; baseline (speedup 1.0000x reference)
import jax
import jax.numpy as jnp
from jax.experimental import pallas as pl


def kernel(x, edge_index, Wp1, bp1, Wl1, bl1, Wr1, Wp2, bp2, Wl2, bl2, Wr2):
    raise NotImplementedError("write your pallas kernel here")


# speedup vs baseline: 1.3401x; 1.3401x over previous
import functools

import jax
import jax.numpy as jnp
from jax import lax
from jax.experimental import pallas as pl
from jax.experimental.pallas import tpu as pltpu

N = 10000
E = 160000
D = 256
H = 512

ROWS = 1000


def _dot_t(a, b):
    return lax.dot_general(a, b, (((1,), (1,)), ((), ())),
                           preferred_element_type=jnp.float32)


def _stage1_body(x_ref, wp1_ref, bp1_ref, wr1_ref, h1_ref, xr1_ref):
    xb = x_ref[...]
    h1_ref[...] = jnp.maximum(_dot_t(xb, wp1_ref[...]) + bp1_ref[...], 0.0)
    xr1_ref[...] = _dot_t(xb, wr1_ref[...])


def _stage1(x, Wp1, bp1, Wr1):
    grid = N // ROWS
    return pl.pallas_call(
        _stage1_body,
        grid=(grid,),
        in_specs=[
            pl.BlockSpec((ROWS, D), lambda i: (i, 0)),
            pl.BlockSpec((D, D), lambda i: (0, 0)),
            pl.BlockSpec((1, D), lambda i: (0, 0)),
            pl.BlockSpec((H, D), lambda i: (0, 0)),
        ],
        out_specs=[
            pl.BlockSpec((ROWS, D), lambda i: (i, 0)),
            pl.BlockSpec((ROWS, H), lambda i: (i, 0)),
        ],
        out_shape=[
            jax.ShapeDtypeStruct((N, D), jnp.float32),
            jax.ShapeDtypeStruct((N, H), jnp.float32),
        ],
    )(x, Wp1, bp1.reshape(1, D), Wr1)


def _stage2_body(ssum_ref, cnt_ref, xr1_ref, wl1_ref, bl1_ref, wp2_ref,
                 bp2_ref, wl2_ref, wr2_ref, s2_ref, r2_ref):
    inv = 1.0 / jnp.maximum(cnt_ref[...], 1.0)
    agg = ssum_ref[...] * inv
    out1 = jnp.maximum(_dot_t(agg, wl1_ref[...]) + bl1_ref[...]
                       + xr1_ref[...], 0.0)
    h2 = jnp.maximum(_dot_t(out1, wp2_ref[...]) + bp2_ref[...], 0.0)
    s2_ref[...] = _dot_t(h2, wl2_ref[...])
    r2_ref[...] = _dot_t(out1, wr2_ref[...])


def _stage2(ssum1, cnt, xr1, Wl1, bl1, Wp2, bp2, Wl2, Wr2):
    grid = N // ROWS
    return pl.pallas_call(
        _stage2_body,
        grid=(grid,),
        in_specs=[
            pl.BlockSpec((ROWS, D), lambda i: (i, 0)),
            pl.BlockSpec((ROWS, 1), lambda i: (i, 0)),
            pl.BlockSpec((ROWS, H), lambda i: (i, 0)),
            pl.BlockSpec((H, D), lambda i: (0, 0)),
            pl.BlockSpec((1, H), lambda i: (0, 0)),
            pl.BlockSpec((H, H), lambda i: (0, 0)),
            pl.BlockSpec((1, H), lambda i: (0, 0)),
            pl.BlockSpec((1, H), lambda i: (0, 0)),
            pl.BlockSpec((1, H), lambda i: (0, 0)),
        ],
        out_specs=[
            pl.BlockSpec((ROWS, 1), lambda i: (i, 0)),
            pl.BlockSpec((ROWS, 1), lambda i: (i, 0)),
        ],
        out_shape=[
            jax.ShapeDtypeStruct((N, 1), jnp.float32),
            jax.ShapeDtypeStruct((N, 1), jnp.float32),
        ],
    )(ssum1, cnt.reshape(N, 1), xr1, Wl1, bl1.reshape(1, H), Wp2,
      bp2.reshape(1, H), Wl2, Wr2)


def kernel(x, edge_index, Wp1, bp1, Wl1, bl1, Wr1, Wp2, bp2, Wl2, bl2, Wr2):
    src = edge_index[0]
    dst = edge_index[1]
    h1, xr1 = _stage1(x, Wp1, bp1, Wr1)
    ssum1 = jax.ops.segment_sum(jnp.take(h1, src, axis=0), dst,
                                num_segments=N)
    cnt = jax.ops.segment_sum(jnp.ones((E,), jnp.float32), dst,
                              num_segments=N)
    s2, r2 = _stage2(ssum1, cnt, xr1, Wl1, bl1, Wp2, bp2, Wl2, Wr2)
    ssum2 = jax.ops.segment_sum(jnp.take(s2[:, 0], src), dst, num_segments=N)
    agg2 = ssum2 / jnp.maximum(cnt, 1.0)
    return jax.nn.sigmoid(agg2[:, None] + bl2 + r2)


# speedup vs baseline: 5.3201x; 3.9700x over previous
import functools

import jax
import jax.numpy as jnp
from jax import lax
from jax.experimental import pallas as pl
from jax.experimental.pallas import tpu as pltpu
from jax.experimental.pallas import tpu_sc as plsc

N = 10000
E = 160000
D = 256
H = 512

ROWS = 1000

EB = 128
NT = 16
EPAD = 163840
TILE_E = EPAD // NT
TILE_B = TILE_E // EB
NPAD = 10240
ROWS_T = NPAD // NT

_MESH = plsc.VectorSubcoreMesh(core_axis_name="c", subcore_axis_name="s")


def _dot_t(a, b):
    return lax.dot_general(a, b, (((1,), (1,)), ((), ())),
                           preferred_element_type=jnp.float32)


def _stage1_body(x_ref, wp1_ref, bp1_ref, wr1_ref, h1_ref, xr1_ref):
    xb = x_ref[...]
    h1_ref[...] = jnp.maximum(_dot_t(xb, wp1_ref[...]) + bp1_ref[...], 0.0)
    xr1_ref[...] = _dot_t(xb, wr1_ref[...])


def _stage1(x, Wp1, bp1, Wr1):
    return pl.pallas_call(
        _stage1_body,
        grid=(N // ROWS,),
        in_specs=[
            pl.BlockSpec((ROWS, D), lambda i: (i, 0)),
            pl.BlockSpec((D, D), lambda i: (0, 0)),
            pl.BlockSpec((1, D), lambda i: (0, 0)),
            pl.BlockSpec((H, D), lambda i: (0, 0)),
        ],
        out_specs=[
            pl.BlockSpec((ROWS, D), lambda i: (i, 0)),
            pl.BlockSpec((ROWS, H), lambda i: (i, 0)),
        ],
        out_shape=[
            jax.ShapeDtypeStruct((N, D), jnp.float32),
            jax.ShapeDtypeStruct((N, H), jnp.float32),
        ],
    )(x, Wp1, bp1.reshape(1, D), Wr1)


def _agg1_body(h_hbm, src_hbm, dst_hbm, ssum_hbm, cnt_hbm,
               zb, zcnt, src_v, dst_v, gidx_v, ones_v, rows_v,
               acc_sh, cnt_sh, sem):
    c = lax.axis_index("c")
    s = lax.axis_index("s")
    zero16 = jnp.zeros((16,), jnp.float32)

    def _zrow(r, _):
        def _zcol(j, _):
            zb[r, pl.ds(j * 16, 16)] = zero16
            return 0
        return lax.fori_loop(0, 8, _zcol, 0)
    lax.fori_loop(0, 16, _zrow, 0)

    def _zc(j, _):
        zcnt[pl.ds(j * 16, 16)] = zero16
        ones_v[pl.ds(j * 16, 16)] = zero16 + 1.0
        return 0
    lax.fori_loop(0, 8, _zc, 0)

    def _zc2(j, _):
        zcnt[pl.ds(128 + j * 16, 16)] = zero16
        return 0
    lax.fori_loop(0, (ROWS_T - 128) // 16, _zc2, 0)

    def _zacc(i, _):
        pltpu.sync_copy(zb, acc_sh.at[pl.ds(s * ROWS_T + i * 16, 16), :])
        return 0
    lax.fori_loop(0, ROWS_T // 16, _zacc, 0)
    pltpu.sync_copy(zcnt, cnt_sh.at[pl.ds(s * ROWS_T, ROWS_T)])
    plsc.subcore_barrier()

    def _edge_batch(b, _):
        base = s * TILE_E + b * EB
        pltpu.sync_copy(src_hbm.at[pl.ds(base, EB)], src_v)
        pltpu.sync_copy(dst_hbm.at[pl.ds(base, EB)], dst_v)

        def _gidx(j, _):
            v = src_v[pl.ds(j * 16, 16)]
            gidx_v[pl.ds(j * 16, 16)] = v * 2 + c
            return 0
        lax.fori_loop(0, EB // 16, _gidx, 0)
        pltpu.async_copy(h_hbm.at[gidx_v], rows_v, sem).wait()
        pltpu.sync_copy(rows_v, acc_sh.at[dst_v], add=True)

        @pl.when(c == 0)
        def _():
            pltpu.sync_copy(ones_v, cnt_sh.at[dst_v], add=True)
        return 0
    lax.fori_loop(0, TILE_B, _edge_batch, 0)
    plsc.subcore_barrier()

    pltpu.sync_copy(acc_sh.at[pl.ds(s * ROWS_T, ROWS_T), :],
                    ssum_hbm.at[c, pl.ds(s * ROWS_T, ROWS_T), :])

    @pl.when(c == 0)
    def _():
        pltpu.sync_copy(cnt_sh.at[pl.ds(s * ROWS_T, ROWS_T)],
                        cnt_hbm.at[pl.ds(s * ROWS_T, ROWS_T)])


_agg1 = functools.partial(
    pl.kernel, _agg1_body, mesh=_MESH,
    out_type=[
        jax.ShapeDtypeStruct((2, NPAD, 128), jnp.float32),
        jax.ShapeDtypeStruct((NPAD,), jnp.float32),
    ],
    scratch_types=[
        pltpu.VMEM((16, 128), jnp.float32),
        pltpu.VMEM((ROWS_T,), jnp.float32),
        pltpu.VMEM((EB,), jnp.int32),
        pltpu.VMEM((EB,), jnp.int32),
        pltpu.VMEM((EB,), jnp.int32),
        pltpu.VMEM((EB,), jnp.float32),
        pltpu.VMEM((EB, 128), jnp.float32),
        pltpu.VMEM_SHARED((NPAD, 128), jnp.float32),
        pltpu.VMEM_SHARED((NPAD,), jnp.float32),
        pltpu.SemaphoreType.DMA,
    ],
)()


def _stage2_body(sa_ref, sb_ref, cnt_ref, xr1_ref, wl1a_ref, wl1b_ref,
                 bl1_ref, wp2_ref, bp2_ref, wl2_ref, wr2_ref,
                 s2_ref, r2_ref):
    inv = 1.0 / jnp.maximum(cnt_ref[...], 1.0)
    lsum = _dot_t(sa_ref[...], wl1a_ref[...]) + _dot_t(sb_ref[...],
                                                       wl1b_ref[...])
    out1 = jnp.maximum(lsum * inv + bl1_ref[...] + xr1_ref[...], 0.0)
    h2 = jnp.maximum(_dot_t(out1, wp2_ref[...]) + bp2_ref[...], 0.0)
    s2_ref[...] = _dot_t(h2, wl2_ref[...])
    r2_ref[...] = _dot_t(out1, wr2_ref[...])


def _stage2(sa, sb, cnt, xr1, Wl1, bl1, Wp2, bp2, Wl2, Wr2):
    return pl.pallas_call(
        _stage2_body,
        grid=(N // ROWS,),
        in_specs=[
            pl.BlockSpec((ROWS, 128), lambda i: (i, 0)),
            pl.BlockSpec((ROWS, 128), lambda i: (i, 0)),
            pl.BlockSpec((ROWS, 1), lambda i: (i, 0)),
            pl.BlockSpec((ROWS, H), lambda i: (i, 0)),
            pl.BlockSpec((H, 128), lambda i: (0, 0)),
            pl.BlockSpec((H, 128), lambda i: (0, 0)),
            pl.BlockSpec((1, H), lambda i: (0, 0)),
            pl.BlockSpec((H, H), lambda i: (0, 0)),
            pl.BlockSpec((1, H), lambda i: (0, 0)),
            pl.BlockSpec((1, H), lambda i: (0, 0)),
            pl.BlockSpec((1, H), lambda i: (0, 0)),
        ],
        out_specs=[
            pl.BlockSpec((ROWS, 1), lambda i: (i, 0)),
            pl.BlockSpec((ROWS, 1), lambda i: (i, 0)),
        ],
        out_shape=[
            jax.ShapeDtypeStruct((N, 1), jnp.float32),
            jax.ShapeDtypeStruct((N, 1), jnp.float32),
        ],
    )(sa, sb, cnt.reshape(N, 1), xr1, Wl1[:, :128], Wl1[:, 128:],
      bl1.reshape(1, H), Wp2, bp2.reshape(1, H), Wl2, Wr2)


def _agg2_body(s2_hbm, src_hbm, dst_hbm, cnt_hbm, r2_hbm, out_hbm,
               zcnt, s2_v, src_v, dst_v, vals_v, a_v, c_v, r_v, o_v,
               acc_sh):
    c = lax.axis_index("c")
    s = lax.axis_index("s")
    zero16 = jnp.zeros((16,), jnp.float32)

    @pl.when(c == 0)
    def _():
        def _zc(j, _):
            zcnt[pl.ds(j * 16, 16)] = zero16
            return 0
        lax.fori_loop(0, ROWS_T // 16, _zc, 0)
        pltpu.sync_copy(zcnt, acc_sh.at[pl.ds(s * ROWS_T, ROWS_T)])
        pltpu.sync_copy(s2_hbm, s2_v)
    plsc.subcore_barrier()

    @pl.when(c == 0)
    def _():
        def _edge_batch(b, _):
            base = s * TILE_E + b * EB
            pltpu.sync_copy(src_hbm.at[pl.ds(base, EB)], src_v)
            pltpu.sync_copy(dst_hbm.at[pl.ds(base, EB)], dst_v)

            def _gather(j, _):
                idx16 = src_v[pl.ds(j * 16, 16)]
                row16 = lax.shift_right_logical(idx16, 7)
                col16 = lax.bitwise_and(idx16, 127)
                vals_v[pl.ds(j * 16, 16)] = plsc.load_gather(
                    s2_v, [row16, col16])
                return 0
            lax.fori_loop(0, EB // 16, _gather, 0)
            pltpu.sync_copy(vals_v, acc_sh.at[dst_v], add=True)
            return 0
        lax.fori_loop(0, TILE_B, _edge_batch, 0)
    plsc.subcore_barrier()

    @pl.when(c == 0)
    def _():
        pltpu.sync_copy(acc_sh.at[pl.ds(s * ROWS_T, ROWS_T)], a_v)
        pltpu.sync_copy(cnt_hbm.at[pl.ds(s * ROWS_T, ROWS_T)], c_v)
        pltpu.sync_copy(r2_hbm.at[pl.ds(s * ROWS_T, ROWS_T)], r_v)

        def _fin(j, _):
            sl = pl.ds(j * 16, 16)
            z = a_v[sl] / jnp.maximum(c_v[sl], 1.0) + r_v[sl]
            o_v[sl] = 1.0 / (1.0 + jnp.exp(-z))
            return 0
        lax.fori_loop(0, ROWS_T // 16, _fin, 0)
        pltpu.sync_copy(o_v, out_hbm.at[pl.ds(s * ROWS_T, ROWS_T)])


_agg2 = functools.partial(
    pl.kernel, _agg2_body, mesh=_MESH,
    compiler_params=pltpu.CompilerParams(needs_layout_passes=False),
    out_type=jax.ShapeDtypeStruct((NPAD,), jnp.float32),
    scratch_types=[
        pltpu.VMEM((ROWS_T,), jnp.float32),
        pltpu.VMEM((NPAD // 128, 128), jnp.float32),
        pltpu.VMEM((EB,), jnp.int32),
        pltpu.VMEM((EB,), jnp.int32),
        pltpu.VMEM((EB,), jnp.float32),
        pltpu.VMEM((ROWS_T,), jnp.float32),
        pltpu.VMEM((ROWS_T,), jnp.float32),
        pltpu.VMEM((ROWS_T,), jnp.float32),
        pltpu.VMEM((ROWS_T,), jnp.float32),
        pltpu.VMEM_SHARED((NPAD,), jnp.float32),
    ],
)()


def kernel(x, edge_index, Wp1, bp1, Wl1, bl1, Wr1, Wp2, bp2, Wl2, bl2, Wr2):
    src = edge_index[0]
    dst = edge_index[1]
    pad = EPAD - E
    src_p = jnp.concatenate([src, jnp.zeros((pad,), jnp.int32)])
    dst_p = jnp.concatenate([dst, jnp.full((pad,), N, jnp.int32)])

    h1, xr1 = _stage1(x, Wp1, bp1, Wr1)
    ssum, cnt = _agg1(h1.reshape(2 * N, 128), src_p, dst_p)
    s2, r2 = _stage2(ssum[0, :N, :], ssum[1, :N, :], cnt[:N], xr1,
                     Wl1, bl1, Wp2, bp2, Wl2, Wr2)
    zpad = jnp.zeros((NPAD - N,), jnp.float32)
    s2_p = jnp.concatenate([s2.reshape(N), zpad])
    r2_p = jnp.concatenate([r2.reshape(N) + bl2[0], zpad])
    out = _agg2(s2_p.reshape(NPAD // 128, 128), src_p, dst_p, cnt, r2_p)
    return out[:N].reshape(N, 1)


# speedup vs baseline: 7.9286x; 1.4903x over previous
import functools

import jax
import jax.numpy as jnp
from jax import lax
from jax.experimental import pallas as pl
from jax.experimental.pallas import tpu as pltpu
from jax.experimental.pallas import tpu_sc as plsc

N = 10000
E = 160000
D = 256
H = 512

ROWS = 1000

EB = 128
NT = 16
EPAD = 163840
TILE_E = EPAD // NT
TILE_B = TILE_E // EB
NPAD = 10240
ROWS_T = NPAD // NT
NBUF = 5

_MESH = plsc.VectorSubcoreMesh(core_axis_name="c", subcore_axis_name="s")


def _dot_t(a, b):
    return lax.dot_general(a, b, (((1,), (1,)), ((), ())),
                           preferred_element_type=jnp.float32)


def _stage1_body(x_ref, wp1_ref, bp1_ref, wr1_ref, h1_ref, xr1_ref):
    xb = x_ref[...]
    h1_ref[...] = jnp.maximum(_dot_t(xb, wp1_ref[...]) + bp1_ref[...], 0.0)
    xr1_ref[...] = _dot_t(xb, wr1_ref[...])


def _stage1(x, Wp1, bp1, Wr1):
    return pl.pallas_call(
        _stage1_body,
        grid=(N // ROWS,),
        in_specs=[
            pl.BlockSpec((ROWS, D), lambda i: (i, 0)),
            pl.BlockSpec((D, D), lambda i: (0, 0)),
            pl.BlockSpec((1, D), lambda i: (0, 0)),
            pl.BlockSpec((H, D), lambda i: (0, 0)),
        ],
        out_specs=[
            pl.BlockSpec((ROWS, D), lambda i: (i, 0)),
            pl.BlockSpec((ROWS, H), lambda i: (i, 0)),
        ],
        out_shape=[
            jax.ShapeDtypeStruct((N, D), jnp.float32),
            jax.ShapeDtypeStruct((N, H), jnp.float32),
        ],
    )(x, Wp1, bp1.reshape(1, D), Wr1)


G = 10
NGRP = TILE_B // G


def _agg1_body(h_hbm, idx_hbm, ssum_hbm, cnt_hbm,
               zb, ones_v, ib0, ib1, rows0, rows1,
               acc_sh, cnt_sh, sem_i, sem_g, sem_s, sem_c, sem_z):
    c = lax.axis_index("c")
    s = lax.axis_index("s")
    rows = (rows0, rows1)
    ibs = (ib0, ib1)
    zero16 = jnp.zeros((16,), jnp.float32)

    def _zrow(r, _):
        def _zcol(j, _):
            zb[r, pl.ds(j * 16, 16)] = zero16
            return 0
        return lax.fori_loop(0, 8, _zcol, 0)
    lax.fori_loop(0, 32, _zrow, 0)

    def _ones(j, _):
        ones_v[pl.ds(j * 16, 16)] = zero16 + 1.0
        return 0
    lax.fori_loop(0, EB // 16, _ones, 0)

    def _idx_start(o, q):
        pltpu.async_copy(idx_hbm.at[c, pl.ds(s * TILE_B + o * G, G), :, :],
                         ibs[q], sem_i.at[q])

    def _idx_wait(o, q):
        pltpu.make_async_copy(
            idx_hbm.at[c, pl.ds(s * TILE_B + o * G, G), :, :],
            ibs[q], sem_i.at[q]).wait()

    def _gather_start(q, i, slot):
        pltpu.async_copy(h_hbm.at[ibs[q].at[i, 0]], rows[slot],
                         sem_g.at[slot])

    def _gather_wait(q, i, slot):
        pltpu.make_async_copy(h_hbm.at[ibs[q].at[i, 0]], rows[slot],
                              sem_g.at[slot]).wait()

    _idx_start(jnp.int32(0), 0)

    for i in range(ROWS_T // 32):
        pltpu.async_copy(zb, acc_sh.at[pl.ds(s * ROWS_T + i * 32, 32), :],
                         sem_z)
    for i in range(ROWS_T // 128):
        pltpu.async_copy(zb.at[0, :],
                         cnt_sh.at[pl.ds(s * ROWS_T + i * 128, 128)], sem_z)
    for i in range(ROWS_T // 32):
        pltpu.make_async_copy(zb, acc_sh.at[pl.ds(s * ROWS_T, 32), :],
                              sem_z).wait()
    for i in range(ROWS_T // 128):
        pltpu.make_async_copy(zb.at[0, :], cnt_sh.at[pl.ds(s * ROWS_T, 128)],
                              sem_z).wait()
    _idx_wait(jnp.int32(0), 0)
    _idx_start(jnp.int32(1), 1)
    _gather_start(0, 0, 0)
    _gather_start(0, 1, 1)
    plsc.subcore_barrier()

    def _outer(oo, _):
        for q in range(2):
            o = oo * 2 + q
            g0 = o * G
            for i in range(G):
                si = i % 2
                g = g0 + i
                _gather_wait(q, i, si)
                pltpu.async_copy(rows[si], acc_sh.at[ibs[q].at[i, 1]],
                                 sem_s.at[si], add=True)

                @pl.when(c == 0)
                def _():
                    @pl.when(g >= 4)
                    def _():
                        pltpu.make_async_copy(
                            ones_v, cnt_sh.at[ibs[q].at[i, 1]],
                            sem_c).wait()
                    pltpu.async_copy(ones_v, cnt_sh.at[ibs[q].at[i, 1]],
                                     sem_c, add=True)

                pltpu.make_async_copy(rows[si], acc_sh.at[ibs[q].at[i, 1]],
                                      sem_s.at[si]).wait()
                if i < G - 2:
                    _gather_start(q, i + 2, si)

            @pl.when(o < NGRP - 1)
            def _():
                _idx_wait(o + 1, 1 - q)
                _gather_start(1 - q, 0, 0)
                _gather_start(1 - q, 1, 1)

                @pl.when(o < NGRP - 2)
                def _():
                    _idx_start(o + 2, q)
        return 0
    lax.fori_loop(0, NGRP // 2, _outer, 0)

    @pl.when(c == 0)
    def _():
        for i in range(4):
            pltpu.make_async_copy(ones_v,
                                  cnt_sh.at[ibs[0].at[jnp.int32(i), 1]],
                                  sem_c).wait()
    plsc.subcore_barrier()

    pltpu.sync_copy(acc_sh.at[pl.ds(s * ROWS_T, ROWS_T), :],
                    ssum_hbm.at[c, pl.ds(s * ROWS_T, ROWS_T), :])

    @pl.when(c == 0)
    def _():
        pltpu.sync_copy(cnt_sh.at[pl.ds(s * ROWS_T, ROWS_T)],
                        cnt_hbm.at[pl.ds(s * ROWS_T, ROWS_T)])


_agg1 = functools.partial(
    pl.kernel, _agg1_body, mesh=_MESH,
    out_type=[
        jax.ShapeDtypeStruct((2, NPAD, 128), jnp.float32),
        jax.ShapeDtypeStruct((NPAD,), jnp.float32),
    ],
    scratch_types=[
        pltpu.VMEM((32, 128), jnp.float32),
        pltpu.VMEM((EB,), jnp.float32),
        pltpu.VMEM((G, 2, EB), jnp.int32),
        pltpu.VMEM((G, 2, EB), jnp.int32),
        pltpu.VMEM((EB, 128), jnp.float32),
        pltpu.VMEM((EB, 128), jnp.float32),
        pltpu.VMEM_SHARED((NPAD, 128), jnp.float32),
        pltpu.VMEM_SHARED((NPAD,), jnp.float32),
        pltpu.SemaphoreType.DMA((2,)),
        pltpu.SemaphoreType.DMA((2,)),
        pltpu.SemaphoreType.DMA((2,)),
        pltpu.SemaphoreType.DMA,
        pltpu.SemaphoreType.DMA,
    ],
)()


def _stage2_body(sa_ref, sb_ref, cnt_ref, xr1_ref, wl1a_ref, wl1b_ref,
                 bl1_ref, wp2_ref, bp2_ref, wl2_ref, wr2_ref,
                 s2_ref, r2_ref):
    inv = 1.0 / jnp.maximum(cnt_ref[...], 1.0)
    lsum = _dot_t(sa_ref[...], wl1a_ref[...]) + _dot_t(sb_ref[...],
                                                       wl1b_ref[...])
    out1 = jnp.maximum(lsum * inv + bl1_ref[...] + xr1_ref[...], 0.0)
    h2 = jnp.maximum(_dot_t(out1, wp2_ref[...]) + bp2_ref[...], 0.0)
    s2_ref[...] = _dot_t(h2, wl2_ref[...])
    r2_ref[...] = _dot_t(out1, wr2_ref[...])


def _stage2(sa, sb, cnt, xr1, Wl1, bl1, Wp2, bp2, Wl2, Wr2):
    return pl.pallas_call(
        _stage2_body,
        grid=(N // ROWS,),
        in_specs=[
            pl.BlockSpec((ROWS, 128), lambda i: (i, 0)),
            pl.BlockSpec((ROWS, 128), lambda i: (i, 0)),
            pl.BlockSpec((ROWS, 1), lambda i: (i, 0)),
            pl.BlockSpec((ROWS, H), lambda i: (i, 0)),
            pl.BlockSpec((H, 128), lambda i: (0, 0)),
            pl.BlockSpec((H, 128), lambda i: (0, 0)),
            pl.BlockSpec((1, H), lambda i: (0, 0)),
            pl.BlockSpec((H, H), lambda i: (0, 0)),
            pl.BlockSpec((1, H), lambda i: (0, 0)),
            pl.BlockSpec((1, H), lambda i: (0, 0)),
            pl.BlockSpec((1, H), lambda i: (0, 0)),
        ],
        out_specs=[
            pl.BlockSpec((ROWS, 1), lambda i: (i, 0)),
            pl.BlockSpec((ROWS, 1), lambda i: (i, 0)),
        ],
        out_shape=[
            jax.ShapeDtypeStruct((N, 1), jnp.float32),
            jax.ShapeDtypeStruct((N, 1), jnp.float32),
        ],
    )(sa, sb, cnt.reshape(N, 1), xr1, Wl1[:, :128], Wl1[:, 128:],
      bl1.reshape(1, H), Wp2, bp2.reshape(1, H), Wl2, Wr2)


def _agg2_body(s2_hbm, src_hbm, dst_hbm, cnt_hbm, r2_hbm, out_hbm,
               zcnt, s2_v, src3, dst3, vals0, vals1, vals2, vals3, vals4,
               a_v, c_v, r_v, o_v, acc_sh, sem_g, sem_s):
    c = lax.axis_index("c")
    s = lax.axis_index("s")
    vals = (vals0, vals1, vals2, vals3, vals4)
    zero16 = jnp.zeros((16,), jnp.float32)

    @pl.when(c == 0)
    def _():
        def _zc(j, _):
            zcnt[pl.ds(j * 16, 16)] = zero16
            return 0
        lax.fori_loop(0, ROWS_T // 16, _zc, 0)
        pltpu.async_copy(src_hbm.at[pl.ds(s * TILE_B, TILE_B), :], src3,
                         sem_g.at[0])
        pltpu.async_copy(dst_hbm.at[pl.ds(s * TILE_B, TILE_B), :], dst3,
                         sem_g.at[1])
        pltpu.async_copy(s2_hbm, s2_v, sem_g.at[2])
        pltpu.sync_copy(zcnt, acc_sh.at[pl.ds(s * ROWS_T, ROWS_T)])
        pltpu.make_async_copy(src_hbm.at[pl.ds(s * TILE_B, TILE_B), :],
                              src3, sem_g.at[0]).wait()
        pltpu.make_async_copy(dst_hbm.at[pl.ds(s * TILE_B, TILE_B), :],
                              dst3, sem_g.at[1]).wait()
        pltpu.make_async_copy(s2_hbm, s2_v, sem_g.at[2]).wait()
    plsc.subcore_barrier()

    @pl.when(c == 0)
    def _():
        def _scat_wait(g, slot):
            pltpu.make_async_copy(vals[slot], acc_sh.at[dst3.at[g]],
                                  sem_s.at[slot]).wait()

        def _group(o, _):
            for i in range(NBUF):
                g = o * NBUF + i

                @pl.when(g >= NBUF)
                def _():
                    _scat_wait(g, i)

                def _gather(j, _):
                    idx16 = src3[g, pl.ds(j * 16, 16)]
                    row16 = lax.shift_right_logical(idx16, 7)
                    col16 = lax.bitwise_and(idx16, 127)
                    vals[i][pl.ds(j * 16, 16)] = plsc.load_gather(
                        s2_v, [row16, col16])
                    return 0
                lax.fori_loop(0, EB // 16, _gather, 0)
                pltpu.async_copy(vals[i], acc_sh.at[dst3.at[g]],
                                 sem_s.at[i], add=True)
            return 0
        lax.fori_loop(0, TILE_B // NBUF, _group, 0)
        for i in range(NBUF):
            _scat_wait(jnp.int32(TILE_B - NBUF + i), i)
    plsc.subcore_barrier()

    @pl.when(c == 0)
    def _():
        pltpu.sync_copy(acc_sh.at[pl.ds(s * ROWS_T, ROWS_T)], a_v)
        pltpu.sync_copy(cnt_hbm.at[pl.ds(s * ROWS_T, ROWS_T)], c_v)
        pltpu.sync_copy(r2_hbm.at[pl.ds(s * ROWS_T, ROWS_T)], r_v)

        def _fin(j, _):
            sl = pl.ds(j * 16, 16)
            z = a_v[sl] / jnp.maximum(c_v[sl], 1.0) + r_v[sl]
            o_v[sl] = 1.0 / (1.0 + jnp.exp(-z))
            return 0
        lax.fori_loop(0, ROWS_T // 16, _fin, 0)
        pltpu.sync_copy(o_v, out_hbm.at[pl.ds(s * ROWS_T, ROWS_T)])


_agg2 = functools.partial(
    pl.kernel, _agg2_body, mesh=_MESH,
    compiler_params=pltpu.CompilerParams(needs_layout_passes=False),
    out_type=jax.ShapeDtypeStruct((NPAD,), jnp.float32),
    scratch_types=[
        pltpu.VMEM((ROWS_T,), jnp.float32),
        pltpu.VMEM((NPAD // 128, 128), jnp.float32),
        pltpu.VMEM((TILE_B, EB), jnp.int32),
        pltpu.VMEM((TILE_B, EB), jnp.int32),
        pltpu.VMEM((EB,), jnp.float32),
        pltpu.VMEM((EB,), jnp.float32),
        pltpu.VMEM((EB,), jnp.float32),
        pltpu.VMEM((EB,), jnp.float32),
        pltpu.VMEM((EB,), jnp.float32),
        pltpu.VMEM((ROWS_T,), jnp.float32),
        pltpu.VMEM((ROWS_T,), jnp.float32),
        pltpu.VMEM((ROWS_T,), jnp.float32),
        pltpu.VMEM((ROWS_T,), jnp.float32),
        pltpu.VMEM_SHARED((NPAD,), jnp.float32),
        pltpu.SemaphoreType.DMA((NBUF,)),
        pltpu.SemaphoreType.DMA((NBUF,)),
    ],
)()


def kernel(x, edge_index, Wp1, bp1, Wl1, bl1, Wr1, Wp2, bp2, Wl2, bl2, Wr2):
    src = edge_index[0]
    dst = edge_index[1]
    pad = EPAD - E
    src_p = jnp.concatenate([src, jnp.zeros((pad,), jnp.int32)])
    dst_p = jnp.concatenate([dst, jnp.full((pad,), N, jnp.int32)])
    src2 = src_p.reshape(EPAD // EB, EB)
    dst2 = dst_p.reshape(EPAD // EB, EB)
    idxcat = jnp.stack([
        jnp.stack([src2 * 2, dst2], axis=1),
        jnp.stack([src2 * 2 + 1, dst2], axis=1),
    ])

    h1, xr1 = _stage1(x, Wp1, bp1, Wr1)
    ssum, cnt = _agg1(h1.reshape(2 * N, 128), idxcat)
    s2, r2 = _stage2(ssum[0, :N, :], ssum[1, :N, :], cnt[:N], xr1,
                     Wl1, bl1, Wp2, bp2, Wl2, Wr2)
    zpad = jnp.zeros((NPAD - N,), jnp.float32)
    s2_p = jnp.concatenate([s2.reshape(N), zpad])
    r2_p = jnp.concatenate([r2.reshape(N) + bl2[0], zpad])
    out = _agg2(s2_p.reshape(NPAD // 128, 128), src2, dst2, cnt, r2_p)
    return out[:N].reshape(N, 1)
